# baseline (speedup 1.0000x reference)
import functools

import jax
import jax.numpy as jnp
from jax import lax
from jax.experimental import pallas as pl
from jax.experimental.pallas import tpu as pltpu
from jax.experimental.pallas import tpu_sc as plsc

B, T, V = 16, 512, 64
BLANK = V - 1
NEG = -1e30
LANES = 16
JPAD = 640


def _k1_body(x_ref, predsk_ref, len_ref):
    CH = 64
    chunks = []
    for c in range(T // CH):
        x = x_ref[:, c * CH:(c + 1) * CH, :]
        m = jnp.max(x, axis=2, keepdims=True)
        iota_v = lax.broadcasted_iota(jnp.int32, (B, CH, V), 2)
        chunks.append(jnp.min(jnp.where(x >= m, iota_v, V), axis=2))
    preds = jnp.concatenate(chunks, axis=1)
    prev = jnp.concatenate(
        [jnp.full((B, 1), -1, jnp.int32), preds[:, :-1]], axis=1)
    keep = (preds != prev) & (preds != BLANK)
    predsk_ref[...] = jnp.where(keep, preds, -1)
    len_ref[...] = jnp.sum(keep.astype(jnp.int32), axis=1, keepdims=True)


_k1 = pl.pallas_call(
    _k1_body,
    out_shape=(jax.ShapeDtypeStruct((B, T), jnp.int32),
               jax.ShapeDtypeStruct((B, 1), jnp.int32)),
)


def _sc_compact_body(predsk_hbm, labels_hbm, in_v, out_v):
    wid = lax.axis_index("s") * 2 + lax.axis_index("c")

    @pl.when(wid < B)
    def _():
        pltpu.sync_copy(predsk_hbm.at[wid], in_v)
        zeros = jnp.zeros((LANES,), jnp.int32)
        for c in range(T // LANES):
            out_v[pl.ds(c * LANES, LANES)] = zeros

        one = jnp.ones((LANES,), jnp.int32)
        zero = jnp.zeros((LANES,), jnp.int32)

        def body(c, tot):
            p = in_v[pl.ds(c * LANES, LANES)]
            k = p >= 0
            ki = jnp.where(k, one, zero)
            pos = plsc.cumsum(ki) - ki
            plsc.store_scatter(out_v, [pos + tot], p, mask=k)
            return tot + jnp.sum(ki)

        lax.fori_loop(0, T // LANES, body, jnp.int32(0))
        pltpu.sync_copy(out_v, labels_hbm.at[wid])


@functools.cache
def _sc_compact():
    return pl.kernel(
        _sc_compact_body,
        mesh=plsc.VectorSubcoreMesh(core_axis_name="c", subcore_axis_name="s",
                                    num_cores=2),
        out_type=jax.ShapeDtypeStruct((B, T), jnp.int32),
        scratch_types=[pltpu.VMEM((T,), jnp.int32),
                       pltpu.VMEM((T,), jnp.int32)],
        compiler_params=pltpu.CompilerParams(needs_layout_passes=False),
    )


def _lae(x, y):
    m = jnp.maximum(x, y)
    return m + jnp.log(1.0 + jnp.exp(-jnp.abs(x - y)))


def _k3_body(x_ref, labels_ref, len_ref, prob_ref, lp_ref, g_ref):
    CH = 64
    for c in range(T // CH):
        x = x_ref[:, c * CH:(c + 1) * CH, :]
        m = jnp.max(x, axis=2, keepdims=True)
        lse = m + jnp.log(jnp.sum(jnp.exp(x - m), axis=2, keepdims=True))
        lp_ref[:, c * CH:(c + 1) * CH, :] = x - lse

    for b in range(B):
        lab = labels_ref[b, :]
        ext = jnp.concatenate(
            [lab, jnp.full((1,), BLANK, jnp.int32),
             jnp.full((JPAD - T - 1,), -7, jnp.int32)])
        oh = (lax.broadcasted_iota(jnp.int32, (V, JPAD), 0)
              == ext[None, :]).astype(jnp.float32)
        for tc in range(4):
            lpb = lp_ref[b, tc * 128:(tc + 1) * 128, :]
            r = lax.dot_general(lpb, oh, (((1,), (0,)), ((), ())),
                                preferred_element_type=jnp.float32)
            g_ref[pl.ds(tc * 128, 128), b, :] = r

    L = len_ref[...]
    labs = labels_ref[...]
    lab_prev = jnp.concatenate(
        [jnp.full((B, 1), BLANK, jnp.int32), labs[:, :-1]], axis=1)
    je = lax.broadcasted_iota(jnp.int32, (B, T + 1), 1)
    jo = lax.broadcasted_iota(jnp.int32, (B, T), 1)
    valid_e = je <= L
    valid_o = jo < L
    skip_add = jnp.where((labs != BLANK) & (labs != lab_prev), 0.0, NEG)

    negcol = jnp.full((B, 1), NEG, jnp.float32)
    skipB_add = jnp.concatenate([skip_add[:, 1:], negcol], axis=1)

    g0 = g_ref[0]
    a_e = jnp.where(je == 0, g0[:, T:T + 1], NEG)
    a_o = jnp.where((jo == 0) & valid_o, g0[:, 0:1], NEG)
    gl = g_ref[T - 1]
    b_e = jnp.where(je == L, gl[:, T:T + 1], NEG)
    b_o = jnp.where(jo == L - 1, gl[:, :T], NEG)

    def step(i, carry):
        a_e, a_o, b_e, b_o = carry
        go = g_ref[i, :, :T]
        gb = g_ref[i, :, T:T + 1]
        sh = jnp.concatenate([negcol, a_o], axis=1)
        me = jnp.maximum(a_e, sh)
        new_ae = me + jnp.log(jnp.exp(a_e - me) + jnp.exp(sh - me)) + gb
        a2 = sh[:, :T] + skip_add
        ae_s = a_e[:, :T]
        mo = jnp.maximum(jnp.maximum(a_o, ae_s), a2)
        so = jnp.exp(a_o - mo) + jnp.exp(ae_s - mo) + jnp.exp(a2 - mo)
        new_ao = mo + jnp.log(so) + go
        tb = T - 1 - i
        hgo = g_ref[tb, :, :T]
        hgb = g_ref[tb, :, T:T + 1]
        bo_pad = jnp.concatenate([b_o, negcol], axis=1)
        mbe = jnp.maximum(b_e, bo_pad)
        new_be = mbe + jnp.log(jnp.exp(b_e - mbe)
                               + jnp.exp(bo_pad - mbe)) + hgb
        be_s = b_e[:, 1:]
        bo_s = jnp.concatenate([b_o[:, 1:], negcol], axis=1)
        b2 = bo_s + skipB_add
        mbo = jnp.maximum(jnp.maximum(b_o, be_s), b2)
        sbo = jnp.exp(b_o - mbo) + jnp.exp(be_s - mbo) + jnp.exp(b2 - mbo)
        new_bo = mbo + jnp.log(sbo) + hgo
        return new_ae, new_ao, new_be, new_bo

    a_e, a_o, b_e, b_o = lax.fori_loop(
        1, T // 2, step, (a_e, a_o, b_e, b_o))

    bo_pad = jnp.concatenate([b_o, negcol], axis=1)
    cap_e = _lae(b_e, bo_pad)
    be_s = b_e[:, 1:]
    bo_s = jnp.concatenate([b_o[:, 1:], negcol], axis=1)
    b2 = bo_s + skipB_add
    cap_o = _lae(_lae(b_o, be_s), b2)
    x_e = a_e + cap_e
    x_o = a_o + cap_o
    m = jnp.maximum(jnp.max(x_e, axis=1, keepdims=True),
                    jnp.max(x_o, axis=1, keepdims=True))
    s = (jnp.sum(jnp.exp(x_e - m), axis=1, keepdims=True)
         + jnp.sum(jnp.exp(x_o - m), axis=1, keepdims=True))
    tot = m + jnp.log(s)
    prob_ref[...] = jnp.exp(tot)


_k3 = pl.pallas_call(
    _k3_body,
    out_shape=jax.ShapeDtypeStruct((B, 1), jnp.float32),
    scratch_shapes=[pltpu.VMEM((B, T, V), jnp.float32),
                    pltpu.VMEM((T, B, JPAD), jnp.float32)],
)


def kernel(inputs):
    predsk, len2 = _k1(inputs)
    labels = _sc_compact()(predsk)
    prob2 = _k3(inputs, labels, len2)
    return labels, len2.reshape(B), prob2.reshape(B)


# speedup vs baseline: 1.1018x; 1.1018x over previous
import functools

import jax
import jax.numpy as jnp
from jax import lax
from jax.experimental import pallas as pl
from jax.experimental.pallas import tpu as pltpu
from jax.experimental.pallas import tpu_sc as plsc

B, T, V = 16, 512, 64
BLANK = V - 1
NEG = -1e30
LANES = 16
JPAD = 640


def _k1_body(x_ref, predsk_ref, len_ref):
    CH = 64
    chunks = []
    for c in range(T // CH):
        x = x_ref[:, c * CH:(c + 1) * CH, :]
        m = jnp.max(x, axis=2, keepdims=True)
        iota_v = lax.broadcasted_iota(jnp.int32, (B, CH, V), 2)
        chunks.append(jnp.min(jnp.where(x >= m, iota_v, V), axis=2))
    preds = jnp.concatenate(chunks, axis=1)
    prev = jnp.concatenate(
        [jnp.full((B, 1), -1, jnp.int32), preds[:, :-1]], axis=1)
    keep = (preds != prev) & (preds != BLANK)
    predsk_ref[...] = jnp.where(keep, preds, -1)
    len_ref[...] = jnp.sum(keep.astype(jnp.int32), axis=1, keepdims=True)


_k1 = pl.pallas_call(
    _k1_body,
    out_shape=(jax.ShapeDtypeStruct((B, T), jnp.int32),
               jax.ShapeDtypeStruct((B, 1), jnp.int32)),
)


def _sc_compact_body(predsk_hbm, labels_hbm, in_v, out_v):
    wid = lax.axis_index("s") * 2 + lax.axis_index("c")

    @pl.when(wid < B)
    def _():
        pltpu.sync_copy(predsk_hbm.at[wid], in_v)
        zeros = jnp.zeros((LANES,), jnp.int32)
        for c in range(T // LANES):
            out_v[pl.ds(c * LANES, LANES)] = zeros

        one = jnp.ones((LANES,), jnp.int32)
        zero = jnp.zeros((LANES,), jnp.int32)

        def body(c, tot):
            p = in_v[pl.ds(c * LANES, LANES)]
            k = p >= 0
            ki = jnp.where(k, one, zero)
            pos = plsc.cumsum(ki) - ki
            plsc.store_scatter(out_v, [pos + tot], p, mask=k)
            return tot + jnp.sum(ki)

        lax.fori_loop(0, T // LANES, body, jnp.int32(0))
        pltpu.sync_copy(out_v, labels_hbm.at[wid])


@functools.cache
def _sc_compact():
    return pl.kernel(
        _sc_compact_body,
        mesh=plsc.VectorSubcoreMesh(core_axis_name="c", subcore_axis_name="s",
                                    num_cores=2),
        out_type=jax.ShapeDtypeStruct((B, T), jnp.int32),
        scratch_types=[pltpu.VMEM((T,), jnp.int32),
                       pltpu.VMEM((T,), jnp.int32)],
        compiler_params=pltpu.CompilerParams(needs_layout_passes=False),
    )


def _lae(x, y):
    m = jnp.maximum(x, y)
    return m + jnp.log(1.0 + jnp.exp(-jnp.abs(x - y)))


def _k3_body(x_ref, labels_ref, len_ref, prob_ref, lp_ref, g_ref):
    CH = 64
    for c in range(T // CH):
        x = x_ref[:, c * CH:(c + 1) * CH, :]
        m = jnp.max(x, axis=2, keepdims=True)
        lse = m + jnp.log(jnp.sum(jnp.exp(x - m), axis=2, keepdims=True))
        lp_ref[:, c * CH:(c + 1) * CH, :] = x - lse

    for b in range(B):
        lab = labels_ref[b, :]
        ext = jnp.concatenate(
            [lab, jnp.full((1,), BLANK, jnp.int32),
             jnp.full((JPAD - T - 1,), -7, jnp.int32)])
        oh = jnp.where(lax.broadcasted_iota(jnp.int32, (V, JPAD), 0)
                       == ext[None, :], 1.4426950408889634, 0.0)
        for tc in range(4):
            lpb = lp_ref[b, tc * 128:(tc + 1) * 128, :]
            r = lax.dot_general(lpb, oh, (((1,), (0,)), ((), ())),
                                preferred_element_type=jnp.float32)
            g_ref[pl.ds(tc * 128, 128), b, :] = r

    L = len_ref[...]
    labs = labels_ref[...]
    lab_prev = jnp.concatenate(
        [jnp.full((B, 1), BLANK, jnp.int32), labs[:, :-1]], axis=1)
    je = lax.broadcasted_iota(jnp.int32, (B, T + 1), 1)
    jo = lax.broadcasted_iota(jnp.int32, (B, T), 1)
    valid_e = je <= L
    valid_o = jo < L
    skip_add = jnp.where((labs != BLANK) & (labs != lab_prev), 0.0, NEG)

    negcol = jnp.full((B, 1), NEG, jnp.float32)

    g0 = g_ref[0]
    a_e = jnp.where(je == 0, g0[:, T:T + 1], NEG)
    a_o = jnp.where((jo == 0) & valid_o, g0[:, 0:1], NEG)

    def step(t, carry):
        a_e, a_o = carry
        go = g_ref[t, :, :T]
        gb = g_ref[t, :, T:T + 1]
        sh = jnp.concatenate([negcol, a_o], axis=1)
        me = jnp.maximum(a_e, sh)
        new_e = me + jnp.log2(jnp.exp2(a_e - me) + jnp.exp2(sh - me)) + gb
        a2 = sh[:, :T] + skip_add
        ae_s = a_e[:, :T]
        mo = jnp.maximum(jnp.maximum(a_o, ae_s), a2)
        so = jnp.exp2(a_o - mo) + jnp.exp2(ae_s - mo) + jnp.exp2(a2 - mo)
        new_o = mo + jnp.log2(so) + go
        return new_e, new_o

    a_e, a_o = lax.fori_loop(1, T, step, (a_e, a_o), unroll=2)

    a_last = jnp.sum(jnp.where(je == L, a_e, 0.0), axis=1, keepdims=True)
    a_prev = jnp.sum(jnp.where(jo == L - 1, a_o, 0.0), axis=1, keepdims=True)
    m2 = jnp.maximum(a_last, a_prev)
    tot2 = m2 + jnp.log2(jnp.exp2(a_last - m2) + jnp.exp2(a_prev - m2))
    tot = jnp.where(L > 0, tot2, a_last)
    prob_ref[...] = jnp.exp2(tot)


_k3 = pl.pallas_call(
    _k3_body,
    out_shape=jax.ShapeDtypeStruct((B, 1), jnp.float32),
    scratch_shapes=[pltpu.VMEM((B, T, V), jnp.float32),
                    pltpu.VMEM((T, B, JPAD), jnp.float32)],
)


def kernel(inputs):
    predsk, len2 = _k1(inputs)
    labels = _sc_compact()(predsk)
    prob2 = _k3(inputs, labels, len2)
    return labels, len2.reshape(B), prob2.reshape(B)


# speedup vs baseline: 1.1302x; 1.0258x over previous
import functools

import jax
import jax.numpy as jnp
from jax import lax
from jax.experimental import pallas as pl
from jax.experimental.pallas import tpu as pltpu
from jax.experimental.pallas import tpu_sc as plsc

B, T, V = 16, 512, 64
BLANK = V - 1
NEG = -1e30
LANES = 16
JPAD = 640


def _k1_body(x_ref, predsk_ref, len_ref):
    CH = 64
    chunks = []
    for c in range(T // CH):
        x = x_ref[:, c * CH:(c + 1) * CH, :]
        m = jnp.max(x, axis=2, keepdims=True)
        iota_v = lax.broadcasted_iota(jnp.int32, (B, CH, V), 2)
        chunks.append(jnp.min(jnp.where(x >= m, iota_v, V), axis=2))
    preds = jnp.concatenate(chunks, axis=1)
    prev = jnp.concatenate(
        [jnp.full((B, 1), -1, jnp.int32), preds[:, :-1]], axis=1)
    keep = (preds != prev) & (preds != BLANK)
    predsk_ref[...] = jnp.where(keep, preds, -1)
    len_ref[...] = jnp.sum(keep.astype(jnp.int32), axis=1, keepdims=True)


_k1 = pl.pallas_call(
    _k1_body,
    out_shape=(jax.ShapeDtypeStruct((B, T), jnp.int32),
               jax.ShapeDtypeStruct((B, 1), jnp.int32)),
)


def _sc_compact_body(predsk_hbm, labels_hbm, in_v, out_v):
    wid = lax.axis_index("s") * 2 + lax.axis_index("c")

    @pl.when(wid < B)
    def _():
        pltpu.sync_copy(predsk_hbm.at[wid], in_v)
        zeros = jnp.zeros((LANES,), jnp.int32)
        for c in range(T // LANES):
            out_v[pl.ds(c * LANES, LANES)] = zeros

        one = jnp.ones((LANES,), jnp.int32)
        zero = jnp.zeros((LANES,), jnp.int32)

        def body(c, tot):
            p = in_v[pl.ds(c * LANES, LANES)]
            k = p >= 0
            ki = jnp.where(k, one, zero)
            pos = plsc.cumsum(ki) - ki
            plsc.store_scatter(out_v, [pos + tot], p, mask=k)
            return tot + jnp.sum(ki)

        lax.fori_loop(0, T // LANES, body, jnp.int32(0))
        pltpu.sync_copy(out_v, labels_hbm.at[wid])


@functools.cache
def _sc_compact():
    return pl.kernel(
        _sc_compact_body,
        mesh=plsc.VectorSubcoreMesh(core_axis_name="c", subcore_axis_name="s",
                                    num_cores=2),
        out_type=jax.ShapeDtypeStruct((B, T), jnp.int32),
        scratch_types=[pltpu.VMEM((T,), jnp.int32),
                       pltpu.VMEM((T,), jnp.int32)],
        compiler_params=pltpu.CompilerParams(needs_layout_passes=False),
    )


def _lae(x, y):
    m = jnp.maximum(x, y)
    return m + jnp.log(1.0 + jnp.exp(-jnp.abs(x - y)))


def _k3_body(x_ref, labels_ref, len_ref, prob_ref, lp_ref, g_ref):
    CH = 64
    for c in range(T // CH):
        x = x_ref[:, c * CH:(c + 1) * CH, :]
        m = jnp.max(x, axis=2, keepdims=True)
        lse = m + jnp.log(jnp.sum(jnp.exp(x - m), axis=2, keepdims=True))
        lp_ref[:, c * CH:(c + 1) * CH, :] = x - lse

    for b in range(B):
        lab = labels_ref[b, :]
        ext = jnp.concatenate(
            [lab, jnp.full((1,), BLANK, jnp.int32),
             jnp.full((JPAD - T - 1,), -7, jnp.int32)])
        oh = jnp.where(lax.broadcasted_iota(jnp.int32, (V, JPAD), 0)
                       == ext[None, :], 1.4426950408889634, 0.0)
        for tc in range(4):
            lpb = lp_ref[b, tc * 128:(tc + 1) * 128, :]
            r = lax.dot_general(lpb, oh, (((1,), (0,)), ((), ())),
                                preferred_element_type=jnp.float32)
            g_ref[pl.ds(tc * 128, 128), b, :] = r

    L = len_ref[...]
    labs = labels_ref[...]
    lab_prev = jnp.concatenate(
        [jnp.full((B, 1), BLANK, jnp.int32), labs[:, :-1]], axis=1)
    je = lax.broadcasted_iota(jnp.int32, (B, T + 1), 1)
    jo = lax.broadcasted_iota(jnp.int32, (B, T), 1)
    valid_e = je <= L
    valid_o = jo < L
    skip_add = jnp.where((labs != BLANK) & (labs != lab_prev), 0.0, NEG)

    negcol = jnp.full((B, 1), NEG, jnp.float32)

    g0 = g_ref[0]
    a_e = jnp.where(je == 0, g0[:, T:T + 1], NEG)
    a_o = jnp.where((jo == 0) & valid_o, g0[:, 0:1], NEG)

    def step(t, carry):
        a_e, a_o = carry
        go = g_ref[t, :, :T]
        gb = g_ref[t, :, T:T + 1]
        sh = jnp.concatenate([negcol, a_o], axis=1)
        me = jnp.maximum(a_e, sh)
        new_e = me + jnp.log2(jnp.exp2(a_e - me) + jnp.exp2(sh - me)) + gb
        a2 = sh[:, :T] + skip_add
        ae_s = a_e[:, :T]
        mo = jnp.maximum(jnp.maximum(a_o, ae_s), a2)
        so = jnp.exp2(a_o - mo) + jnp.exp2(ae_s - mo) + jnp.exp2(a2 - mo)
        new_o = mo + jnp.log2(so) + go
        return new_e, new_o

    a_e, a_o = lax.fori_loop(1, T, step, (a_e, a_o), unroll=4)

    a_last = jnp.sum(jnp.where(je == L, a_e, 0.0), axis=1, keepdims=True)
    a_prev = jnp.sum(jnp.where(jo == L - 1, a_o, 0.0), axis=1, keepdims=True)
    m2 = jnp.maximum(a_last, a_prev)
    tot2 = m2 + jnp.log2(jnp.exp2(a_last - m2) + jnp.exp2(a_prev - m2))
    tot = jnp.where(L > 0, tot2, a_last)
    prob_ref[...] = jnp.exp2(tot)


_k3 = pl.pallas_call(
    _k3_body,
    out_shape=jax.ShapeDtypeStruct((B, 1), jnp.float32),
    scratch_shapes=[pltpu.VMEM((B, T, V), jnp.float32),
                    pltpu.VMEM((T, B, JPAD), jnp.float32)],
)


def kernel(inputs):
    predsk, len2 = _k1(inputs)
    labels = _sc_compact()(predsk)
    prob2 = _k3(inputs, labels, len2)
    return labels, len2.reshape(B), prob2.reshape(B)


# speedup vs baseline: 1.1444x; 1.0126x over previous
import functools

import jax
import jax.numpy as jnp
from jax import lax
from jax.experimental import pallas as pl
from jax.experimental.pallas import tpu as pltpu
from jax.experimental.pallas import tpu_sc as plsc

B, T, V = 16, 512, 64
BLANK = V - 1
NEG = -1e30
LANES = 16
JPAD = 640


def _k1_body(x_ref, predsk_ref, len_ref):
    CH = 64
    chunks = []
    for c in range(T // CH):
        x = x_ref[:, c * CH:(c + 1) * CH, :]
        m = jnp.max(x, axis=2, keepdims=True)
        iota_v = lax.broadcasted_iota(jnp.int32, (B, CH, V), 2)
        chunks.append(jnp.min(jnp.where(x >= m, iota_v, V), axis=2))
    preds = jnp.concatenate(chunks, axis=1)
    prev = jnp.concatenate(
        [jnp.full((B, 1), -1, jnp.int32), preds[:, :-1]], axis=1)
    keep = (preds != prev) & (preds != BLANK)
    predsk_ref[...] = jnp.where(keep, preds, -1)
    len_ref[...] = jnp.sum(keep.astype(jnp.int32), axis=1, keepdims=True)


_k1 = pl.pallas_call(
    _k1_body,
    out_shape=(jax.ShapeDtypeStruct((B, T), jnp.int32),
               jax.ShapeDtypeStruct((B, 1), jnp.int32)),
)


def _sc_compact_body(predsk_hbm, labels_hbm, in_v, out_v):
    wid = lax.axis_index("s") * 2 + lax.axis_index("c")

    @pl.when(wid < B)
    def _():
        pltpu.sync_copy(predsk_hbm.at[wid], in_v)
        zeros = jnp.zeros((LANES,), jnp.int32)
        for c in range(T // LANES):
            out_v[pl.ds(c * LANES, LANES)] = zeros

        one = jnp.ones((LANES,), jnp.int32)
        zero = jnp.zeros((LANES,), jnp.int32)

        def body(c, tot):
            p = in_v[pl.ds(c * LANES, LANES)]
            k = p >= 0
            ki = jnp.where(k, one, zero)
            pos = plsc.cumsum(ki) - ki
            plsc.store_scatter(out_v, [pos + tot], p, mask=k)
            return tot + jnp.sum(ki)

        lax.fori_loop(0, T // LANES, body, jnp.int32(0))
        pltpu.sync_copy(out_v, labels_hbm.at[wid])


@functools.cache
def _sc_compact():
    return pl.kernel(
        _sc_compact_body,
        mesh=plsc.VectorSubcoreMesh(core_axis_name="c", subcore_axis_name="s",
                                    num_cores=2),
        out_type=jax.ShapeDtypeStruct((B, T), jnp.int32),
        scratch_types=[pltpu.VMEM((T,), jnp.int32),
                       pltpu.VMEM((T,), jnp.int32)],
        compiler_params=pltpu.CompilerParams(needs_layout_passes=False),
    )


def _lae(x, y):
    m = jnp.maximum(x, y)
    return m + jnp.log(1.0 + jnp.exp(-jnp.abs(x - y)))


def _k3_body(x_ref, labels_ref, len_ref, prob_ref, lp_ref, g_ref):
    CH = 64
    for c in range(T // CH):
        x = x_ref[:, c * CH:(c + 1) * CH, :]
        m = jnp.max(x, axis=2, keepdims=True)
        lse = m + jnp.log(jnp.sum(jnp.exp(x - m), axis=2, keepdims=True))
        lp_ref[:, c * CH:(c + 1) * CH, :] = x - lse

    for b in range(B):
        lab = labels_ref[b, :]
        ext = jnp.concatenate(
            [lab, jnp.full((1,), BLANK, jnp.int32),
             jnp.full((JPAD - T - 1,), -7, jnp.int32)])
        oh = jnp.where(lax.broadcasted_iota(jnp.int32, (V, JPAD), 0)
                       == ext[None, :], 1.4426950408889634, 0.0)
        for tc in range(4):
            lpb = lp_ref[b, tc * 128:(tc + 1) * 128, :]
            r = lax.dot_general(lpb, oh, (((1,), (0,)), ((), ())),
                                preferred_element_type=jnp.float32)
            g_ref[pl.ds(tc * 128, 128), b, :] = r

    L = len_ref[...]
    labs = labels_ref[...]
    lab_prev = jnp.concatenate(
        [jnp.full((B, 1), BLANK, jnp.int32), labs[:, :-1]], axis=1)
    je = lax.broadcasted_iota(jnp.int32, (B, T + 1), 1)
    jo = lax.broadcasted_iota(jnp.int32, (B, T), 1)
    valid_e = je <= L
    valid_o = jo < L
    skip_add = jnp.where((labs != BLANK) & (labs != lab_prev), 0.0, NEG)

    negcol = jnp.full((B, 1), NEG, jnp.float32)

    g0 = g_ref[0]
    a_e = jnp.where(je == 0, g0[:, T:T + 1], NEG)
    a_o = jnp.where((jo == 0) & valid_o, g0[:, 0:1], NEG)

    def step(t, carry):
        a_e, a_o = carry
        go = g_ref[t, :, :T]
        gb = g_ref[t, :, T:T + 1]
        sh = jnp.concatenate([negcol, a_o], axis=1)
        me = jnp.maximum(a_e, sh)
        new_e = me + jnp.log2(jnp.exp2(a_e - me) + jnp.exp2(sh - me)) + gb
        a2 = sh[:, :T] + skip_add
        ae_s = a_e[:, :T]
        mo = jnp.maximum(jnp.maximum(a_o, ae_s), a2)
        so = jnp.exp2(a_o - mo) + jnp.exp2(ae_s - mo) + jnp.exp2(a2 - mo)
        new_o = mo + jnp.log2(so) + go
        return new_e, new_o

    a_e, a_o = lax.fori_loop(1, T, step, (a_e, a_o), unroll=8)

    a_last = jnp.sum(jnp.where(je == L, a_e, 0.0), axis=1, keepdims=True)
    a_prev = jnp.sum(jnp.where(jo == L - 1, a_o, 0.0), axis=1, keepdims=True)
    m2 = jnp.maximum(a_last, a_prev)
    tot2 = m2 + jnp.log2(jnp.exp2(a_last - m2) + jnp.exp2(a_prev - m2))
    tot = jnp.where(L > 0, tot2, a_last)
    prob_ref[...] = jnp.exp2(tot)


_k3 = pl.pallas_call(
    _k3_body,
    out_shape=jax.ShapeDtypeStruct((B, 1), jnp.float32),
    scratch_shapes=[pltpu.VMEM((B, T, V), jnp.float32),
                    pltpu.VMEM((T, B, JPAD), jnp.float32)],
)


def kernel(inputs):
    predsk, len2 = _k1(inputs)
    labels = _sc_compact()(predsk)
    prob2 = _k3(inputs, labels, len2)
    return labels, len2.reshape(B), prob2.reshape(B)


# speedup vs baseline: 1.1579x; 1.0118x over previous
import functools

import jax
import jax.numpy as jnp
from jax import lax
from jax.experimental import pallas as pl
from jax.experimental.pallas import tpu as pltpu
from jax.experimental.pallas import tpu_sc as plsc

B, T, V = 16, 512, 64
BLANK = V - 1
NEG = -1e30
LANES = 16
JPAD = 640


def _k1_body(x_ref, predsk_ref, len_ref):
    CH = 64
    chunks = []
    for c in range(T // CH):
        x = x_ref[:, c * CH:(c + 1) * CH, :]
        m = jnp.max(x, axis=2, keepdims=True)
        iota_v = lax.broadcasted_iota(jnp.int32, (B, CH, V), 2)
        chunks.append(jnp.min(jnp.where(x >= m, iota_v, V), axis=2))
    preds = jnp.concatenate(chunks, axis=1)
    prev = jnp.concatenate(
        [jnp.full((B, 1), -1, jnp.int32), preds[:, :-1]], axis=1)
    keep = (preds != prev) & (preds != BLANK)
    predsk_ref[...] = jnp.where(keep, preds, -1)
    len_ref[...] = jnp.sum(keep.astype(jnp.int32), axis=1, keepdims=True)


_k1 = pl.pallas_call(
    _k1_body,
    out_shape=(jax.ShapeDtypeStruct((B, T), jnp.int32),
               jax.ShapeDtypeStruct((B, 1), jnp.int32)),
)


def _sc_compact_body(predsk_hbm, labels_hbm, in_v, out_v):
    wid = lax.axis_index("s") * 2 + lax.axis_index("c")

    @pl.when(wid < B)
    def _():
        pltpu.sync_copy(predsk_hbm.at[wid], in_v)
        zeros = jnp.zeros((LANES,), jnp.int32)
        for c in range(T // LANES):
            out_v[pl.ds(c * LANES, LANES)] = zeros

        one = jnp.ones((LANES,), jnp.int32)
        zero = jnp.zeros((LANES,), jnp.int32)

        def body(c, tot):
            p = in_v[pl.ds(c * LANES, LANES)]
            k = p >= 0
            ki = jnp.where(k, one, zero)
            pos = plsc.cumsum(ki) - ki
            plsc.store_scatter(out_v, [pos + tot], p, mask=k)
            return tot + jnp.sum(ki)

        lax.fori_loop(0, T // LANES, body, jnp.int32(0))
        pltpu.sync_copy(out_v, labels_hbm.at[wid])


@functools.cache
def _sc_compact():
    return pl.kernel(
        _sc_compact_body,
        mesh=plsc.VectorSubcoreMesh(core_axis_name="c", subcore_axis_name="s",
                                    num_cores=2),
        out_type=jax.ShapeDtypeStruct((B, T), jnp.int32),
        scratch_types=[pltpu.VMEM((T,), jnp.int32),
                       pltpu.VMEM((T,), jnp.int32)],
        compiler_params=pltpu.CompilerParams(needs_layout_passes=False),
    )


def _lae(x, y):
    m = jnp.maximum(x, y)
    return m + jnp.log(1.0 + jnp.exp(-jnp.abs(x - y)))


def _k3_body(x_ref, labels_ref, len_ref, prob_ref, lp_ref, g_ref):
    CH = 64
    for c in range(T // CH):
        x = x_ref[:, c * CH:(c + 1) * CH, :]
        m = jnp.max(x, axis=2, keepdims=True)
        lse = m + jnp.log(jnp.sum(jnp.exp(x - m), axis=2, keepdims=True))
        lp_ref[:, c * CH:(c + 1) * CH, :] = x - lse

    for b in range(B):
        lab = labels_ref[b, :]
        ext = jnp.concatenate(
            [lab, jnp.full((1,), BLANK, jnp.int32),
             jnp.full((JPAD - T - 1,), -7, jnp.int32)])
        oh = jnp.where(lax.broadcasted_iota(jnp.int32, (V, JPAD), 0)
                       == ext[None, :], 1.4426950408889634, 0.0)
        for tc in range(4):
            lpb = lp_ref[b, tc * 128:(tc + 1) * 128, :]
            r = lax.dot_general(lpb, oh, (((1,), (0,)), ((), ())),
                                preferred_element_type=jnp.float32)
            g_ref[pl.ds(tc * 128, 128), b, :] = r

    L = len_ref[...]
    labs = labels_ref[...]
    lab_prev = jnp.concatenate(
        [jnp.full((B, 1), BLANK, jnp.int32), labs[:, :-1]], axis=1)
    je = lax.broadcasted_iota(jnp.int32, (B, T + 1), 1)
    jo = lax.broadcasted_iota(jnp.int32, (B, T), 1)
    valid_e = je <= L
    valid_o = jo < L
    skip_add = jnp.where((labs != BLANK) & (labs != lab_prev), 0.0, NEG)

    negcol = jnp.full((B, 1), NEG, jnp.float32)

    g0 = g_ref[0]
    a_e = jnp.where(je == 0, g0[:, T:T + 1], NEG)
    a_o = jnp.where((jo == 0) & valid_o, g0[:, 0:1], NEG)

    def step(t, carry):
        a_e, a_o = carry
        go = g_ref[t, :, :T]
        gb = g_ref[t, :, T:T + 1]
        sh = jnp.concatenate([negcol, a_o], axis=1)
        me = jnp.maximum(a_e, sh)
        new_e = me + jnp.log2(jnp.exp2(a_e - me) + jnp.exp2(sh - me)) + gb
        a2 = sh[:, :T] + skip_add
        ae_s = a_e[:, :T]
        mo = jnp.maximum(jnp.maximum(a_o, ae_s), a2)
        so = jnp.exp2(a_o - mo) + jnp.exp2(ae_s - mo) + jnp.exp2(a2 - mo)
        new_o = mo + jnp.log2(so) + go
        return new_e, new_o

    a_e, a_o = lax.fori_loop(1, T, step, (a_e, a_o), unroll=16)

    a_last = jnp.sum(jnp.where(je == L, a_e, 0.0), axis=1, keepdims=True)
    a_prev = jnp.sum(jnp.where(jo == L - 1, a_o, 0.0), axis=1, keepdims=True)
    m2 = jnp.maximum(a_last, a_prev)
    tot2 = m2 + jnp.log2(jnp.exp2(a_last - m2) + jnp.exp2(a_prev - m2))
    tot = jnp.where(L > 0, tot2, a_last)
    prob_ref[...] = jnp.exp2(tot)


_k3 = pl.pallas_call(
    _k3_body,
    out_shape=jax.ShapeDtypeStruct((B, 1), jnp.float32),
    scratch_shapes=[pltpu.VMEM((B, T, V), jnp.float32),
                    pltpu.VMEM((T, B, JPAD), jnp.float32)],
)


def kernel(inputs):
    predsk, len2 = _k1(inputs)
    labels = _sc_compact()(predsk)
    prob2 = _k3(inputs, labels, len2)
    return labels, len2.reshape(B), prob2.reshape(B)


# speedup vs baseline: 1.1669x; 1.0077x over previous
import functools

import jax
import jax.numpy as jnp
from jax import lax
from jax.experimental import pallas as pl
from jax.experimental.pallas import tpu as pltpu
from jax.experimental.pallas import tpu_sc as plsc

B, T, V = 16, 512, 64
BLANK = V - 1
NEG = -1e30
LANES = 16
JPAD = 640


def _k1_body(x_ref, predsk_ref, len_ref):
    CH = 64
    chunks = []
    for c in range(T // CH):
        x = x_ref[:, c * CH:(c + 1) * CH, :]
        m = jnp.max(x, axis=2, keepdims=True)
        iota_v = lax.broadcasted_iota(jnp.int32, (B, CH, V), 2)
        chunks.append(jnp.min(jnp.where(x >= m, iota_v, V), axis=2))
    preds = jnp.concatenate(chunks, axis=1)
    prev = jnp.concatenate(
        [jnp.full((B, 1), -1, jnp.int32), preds[:, :-1]], axis=1)
    keep = (preds != prev) & (preds != BLANK)
    predsk_ref[...] = jnp.where(keep, preds, -1)
    len_ref[...] = jnp.sum(keep.astype(jnp.int32), axis=1, keepdims=True)


_k1 = pl.pallas_call(
    _k1_body,
    out_shape=(jax.ShapeDtypeStruct((B, T), jnp.int32),
               jax.ShapeDtypeStruct((B, 1), jnp.int32)),
)


def _sc_compact_body(predsk_hbm, labels_hbm, in_v, out_v):
    wid = lax.axis_index("s") * 2 + lax.axis_index("c")

    @pl.when(wid < B)
    def _():
        pltpu.sync_copy(predsk_hbm.at[wid], in_v)
        zeros = jnp.zeros((LANES,), jnp.int32)
        for c in range(T // LANES):
            out_v[pl.ds(c * LANES, LANES)] = zeros

        one = jnp.ones((LANES,), jnp.int32)
        zero = jnp.zeros((LANES,), jnp.int32)

        def body(c, tot):
            p = in_v[pl.ds(c * LANES, LANES)]
            k = p >= 0
            ki = jnp.where(k, one, zero)
            pos = plsc.cumsum(ki) - ki
            plsc.store_scatter(out_v, [pos + tot], p, mask=k)
            return tot + jnp.sum(ki)

        lax.fori_loop(0, T // LANES, body, jnp.int32(0))
        pltpu.sync_copy(out_v, labels_hbm.at[wid])


@functools.cache
def _sc_compact():
    return pl.kernel(
        _sc_compact_body,
        mesh=plsc.VectorSubcoreMesh(core_axis_name="c", subcore_axis_name="s",
                                    num_cores=2),
        out_type=jax.ShapeDtypeStruct((B, T), jnp.int32),
        scratch_types=[pltpu.VMEM((T,), jnp.int32),
                       pltpu.VMEM((T,), jnp.int32)],
        compiler_params=pltpu.CompilerParams(needs_layout_passes=False),
    )


def _lae(x, y):
    m = jnp.maximum(x, y)
    return m + jnp.log(1.0 + jnp.exp(-jnp.abs(x - y)))


def _k3_body(x_ref, labels_ref, len_ref, prob_ref, lp_ref, g_ref):
    CH = 64
    for c in range(T // CH):
        x = x_ref[:, c * CH:(c + 1) * CH, :]
        m = jnp.max(x, axis=2, keepdims=True)
        lse = m + jnp.log(jnp.sum(jnp.exp(x - m), axis=2, keepdims=True))
        lp_ref[:, c * CH:(c + 1) * CH, :] = x - lse

    for b in range(B):
        lab = labels_ref[b, :]
        ext = jnp.concatenate(
            [lab, jnp.full((1,), BLANK, jnp.int32),
             jnp.full((JPAD - T - 1,), -7, jnp.int32)])
        oh = jnp.where(lax.broadcasted_iota(jnp.int32, (V, JPAD), 0)
                       == ext[None, :], 1.4426950408889634, 0.0)
        for tc in range(4):
            lpb = lp_ref[b, tc * 128:(tc + 1) * 128, :]
            r = lax.dot_general(lpb, oh, (((1,), (0,)), ((), ())),
                                preferred_element_type=jnp.float32)
            g_ref[pl.ds(tc * 128, 128), b, :] = r

    L = len_ref[...]
    labs = labels_ref[...]
    lab_prev = jnp.concatenate(
        [jnp.full((B, 1), BLANK, jnp.int32), labs[:, :-1]], axis=1)
    je = lax.broadcasted_iota(jnp.int32, (B, T + 1), 1)
    jo = lax.broadcasted_iota(jnp.int32, (B, T), 1)
    valid_e = je <= L
    valid_o = jo < L
    skip_add = jnp.where((labs != BLANK) & (labs != lab_prev), 0.0, NEG)

    negcol = jnp.full((B, 1), NEG, jnp.float32)

    g0 = g_ref[0]
    a_e = jnp.where(je == 0, g0[:, T:T + 1], NEG)
    a_o = jnp.where((jo == 0) & valid_o, g0[:, 0:1], NEG)

    def step(t, carry):
        a_e, a_o = carry
        go = g_ref[t, :, :T]
        gb = g_ref[t, :, T:T + 1]
        sh = jnp.concatenate([negcol, a_o], axis=1)
        me = jnp.maximum(a_e, sh)
        new_e = me + jnp.log2(jnp.exp2(a_e - me) + jnp.exp2(sh - me)) + gb
        a2 = sh[:, :T] + skip_add
        ae_s = a_e[:, :T]
        mo = jnp.maximum(jnp.maximum(a_o, ae_s), a2)
        so = jnp.exp2(a_o - mo) + jnp.exp2(ae_s - mo) + jnp.exp2(a2 - mo)
        new_o = mo + jnp.log2(so) + go
        return new_e, new_o

    a_e, a_o = lax.fori_loop(1, T, step, (a_e, a_o), unroll=32)

    a_last = jnp.sum(jnp.where(je == L, a_e, 0.0), axis=1, keepdims=True)
    a_prev = jnp.sum(jnp.where(jo == L - 1, a_o, 0.0), axis=1, keepdims=True)
    m2 = jnp.maximum(a_last, a_prev)
    tot2 = m2 + jnp.log2(jnp.exp2(a_last - m2) + jnp.exp2(a_prev - m2))
    tot = jnp.where(L > 0, tot2, a_last)
    prob_ref[...] = jnp.exp2(tot)


_k3 = pl.pallas_call(
    _k3_body,
    out_shape=jax.ShapeDtypeStruct((B, 1), jnp.float32),
    scratch_shapes=[pltpu.VMEM((B, T, V), jnp.float32),
                    pltpu.VMEM((T, B, JPAD), jnp.float32)],
)


def kernel(inputs):
    predsk, len2 = _k1(inputs)
    labels = _sc_compact()(predsk)
    prob2 = _k3(inputs, labels, len2)
    return labels, len2.reshape(B), prob2.reshape(B)
